# baseline (speedup 1.0000x reference)
import functools

import jax
import jax.numpy as jnp
from jax import lax
from jax.experimental import pallas as pl
from jax.experimental.pallas import tpu as pltpu
from jax.experimental.pallas import tpu_sc as plsc

ZD, YD, XD = 41, 80, 80
EPS = 1e-3
F32 = jnp.float32

SC_ROWS = 282624
SC_ZONE = 141312
SC_OFF = 64
SC_NVOX = 40960
SC_PER = 2560
SC_ZPER = 8832


def _sc_scatter(rows, idx0, idx1, zrows):
    mesh = plsc.VectorSubcoreMesh(core_axis_name="c", subcore_axis_name="s",
                                  num_cores=2)

    @functools.partial(
        pl.kernel, mesh=mesh,
        out_type=jax.ShapeDtypeStruct((SC_ROWS, 128), F32),
        scratch_types=[
            pltpu.VMEM((SC_PER // 128, 128), jnp.int32),
            pltpu.VMEM((256, 128), F32),
            pltpu.SemaphoreType.DMA,
        ],
    )
    def k(rows_hbm, i0_hbm, i1_hbm, z_hbm, out_hbm, idx_v, rows_v, sem):
        c = lax.axis_index("c")
        s = lax.axis_index("s")
        zstart = c * SC_ZONE + s * SC_ZPER
        pltpu.sync_copy(z_hbm, out_hbm.at[pl.ds(zstart, SC_ZPER)])
        plsc.subcore_barrier()

        @pl.when(c == 0)
        def _():
            pltpu.sync_copy(i0_hbm.at[s], idx_v)

        @pl.when(c == 1)
        def _():
            pltpu.sync_copy(i1_hbm.at[s], idx_v)

        for t in range(SC_PER // 256):
            pltpu.sync_copy(
                rows_hbm.at[pl.ds(s * SC_PER + t * 256, 256)], rows_v)
            c0 = pltpu.async_copy(rows_v.at[pl.ds(0, 128)],
                                  out_hbm.at[idx_v.at[2 * t]], sem)
            c1 = pltpu.async_copy(rows_v.at[pl.ds(128, 128)],
                                  out_hbm.at[idx_v.at[2 * t + 1]], sem)
            c0.wait()
            c1.wait()

    return k(rows, idx0, idx1, zrows)


def _conv_layer(x, m, wT, *, Zi, Zo, s, zoff, Wf, kyx, act, subm,
                sc=None, sh=None, sel=None):
    G = Wf + 1
    P = Wf * Wf
    M = P + 2 * G
    _, Ci, _ = x.shape
    Co = wT.shape[0]
    taps = [(dy, dx) for dy in range(kyx) for dx in range(kyx)]

    def body(*refs):
        xr = refs[0:3]
        mr = refs[3:6]
        i = 6
        if act:
            scr, shr = refs[i], refs[i + 1]
            i += 2
        if sel is not None:
            selr = refs[i]
            i += 1
        wr = refs[i]
        yref, nmref, pref = refs[i + 1:i + 4]

        zi = pl.program_id(0)
        mc = jnp.zeros((1, P), F32)
        m_center = None
        sls = []
        for dz in range(3):
            vz = s * zi + zoff + dz
            valid = jnp.where((vz >= 0) & (vz < Zi), 1.0, 0.0)
            mv = mr[dz][0] * valid
            if dz == 1:
                m_center = mr[dz][0]
            xv = xr[dz][0]
            if act:
                av = jnp.maximum(xv * scr[...] + shr[...], 0.0) * mv
            else:
                av = xv * valid
            for dy, dx in taps:
                o = (dy - 1) * Wf + (dx - 1) if kyx == 3 else 0
                sls.append(jax.lax.slice(av, (0, G + o), (Ci, G + o + P)))
                if not subm:
                    mc = mc + jax.lax.slice(mv, (0, G + o), (1, G + o + P))
        rows = sls[0] if len(sls) == 1 else jnp.concatenate(sls, axis=0)
        acc = jnp.dot(wr[...], rows, preferred_element_type=F32)
        if subm:
            nm_core = jax.lax.slice(m_center, (0, G), (1, G + P))
            nmfull = m_center
        else:
            nm_core = (mc > 0.5).astype(F32)
            nm_core = nm_core * jax.lax.slice(selr[...], (0, G), (1, G + P))
            zg = jnp.zeros((1, G), F32)
            nmfull = jnp.concatenate([zg, nm_core, zg], axis=1)
        zgc = jnp.zeros((Co, G), F32)
        yref[0] = jnp.concatenate([zgc, acc, zgc], axis=1)
        nmref[0] = nmfull

        nm_eff = nm_core
        s1 = jnp.sum(acc * nm_eff, axis=1, keepdims=True)
        s2 = jnp.sum(acc * acc * nm_eff, axis=1, keepdims=True)
        cm = jnp.zeros((Co, 1), F32) + jnp.sum(nm_eff)
        blk = jnp.concatenate([s1, s2, cm, jnp.zeros((Co, 5), F32)], axis=1)
        if Co < 128:
            blk = jnp.concatenate([blk, jnp.zeros((128 - Co, 8), F32)],
                                  axis=0)

        @pl.when(zi == 0)
        def _():
            pref[...] = blk

        @pl.when(zi != 0)
        def _():
            pref[...] = pref[...] + blk

    def xmap(d):
        return lambda z: (jnp.clip(s * z + zoff + d, 0, Zi - 1), 0, 0)

    def mmap(d):
        return lambda z: (jnp.clip(s * z + zoff + d, 0, Zi - 1), 0, 0)

    in_specs = [pl.BlockSpec((1, Ci, M), xmap(d)) for d in range(3)]
    in_specs += [pl.BlockSpec((1, 1, M), mmap(d)) for d in range(3)]
    args = [x, x, x, m, m, m]
    if act:
        in_specs += [pl.BlockSpec((Ci, 1), lambda z: (0, 0))] * 2
        args += [sc, sh]
    if sel is not None:
        in_specs += [pl.BlockSpec((1, M), lambda z: (0, 0))]
        args += [sel]
    in_specs += [pl.BlockSpec(wT.shape, lambda z: (0, 0))]
    args += [wT]

    return pl.pallas_call(
        body,
        grid=(Zo,),
        in_specs=in_specs,
        out_specs=[
            pl.BlockSpec((1, Co, M), lambda z: (z, 0, 0)),
            pl.BlockSpec((1, 1, M), lambda z: (z, 0, 0)),
            pl.BlockSpec((128, 8), lambda z: (0, 0)),
        ],
        out_shape=[
            jax.ShapeDtypeStruct((Zo, Co, M), F32),
            jax.ShapeDtypeStruct((Zo, 1, M), F32),
            jax.ShapeDtypeStruct((128, 8), F32),
        ],
    )(*args)


def _final_apply(y, nm, sc, sh):
    def body(yr, mr, scr, shr, aref):
        a = jnp.maximum(yr[...] * scr[...] + shr[...], 0.0)
        aref[...] = a * mr[...]

    return pl.pallas_call(
        body,
        out_shape=jax.ShapeDtypeStruct(y.shape, F32),
    )(y, nm, sc, sh)


def _finalize(part, g, b):
    Co = g.shape[0]
    cnt = jnp.maximum(part[0, 2], 1.0)
    mean = part[:Co, 0] / cnt
    var = jnp.maximum(part[:Co, 1] / cnt - mean * mean, 0.0)
    scale = g * jax.lax.rsqrt(var + EPS)
    shift = b - mean * scale
    return scale[:, None], shift[:, None]


def _compact(yf, nmf, Whi, Wlo):
    Zo, Co, _ = yf.shape
    Ghi, Phi = Whi + 1, Whi * Whi
    Glo = Wlo + 1
    core = yf[:, :, Ghi:Ghi + Phi].reshape(Zo, Co, Whi, Whi)
    sub = core[:, :, 1:Whi - 1:2, 1:Whi - 1:2]
    y = jnp.pad(sub, ((0, 0), (0, 0), (1, 1), (1, 1)))
    y = y.reshape(Zo, Co, Wlo * Wlo)
    y = jnp.pad(y, ((0, 0), (0, 0), (Glo, Glo)))
    mcore = nmf[:, 0, Ghi:Ghi + Phi].reshape(Zo, Whi, Whi)
    msub = mcore[:, 1:Whi - 1:2, 1:Whi - 1:2]
    nm = jnp.pad(msub, ((0, 0), (1, 1), (1, 1))).reshape(Zo, 1, Wlo * Wlo)
    nm = jnp.pad(nm, ((0, 0), (0, 0), (Glo, Glo)))
    return y, nm


def _selmask(Wf, stride2):
    G = Wf + 1
    i = jnp.arange(Wf)
    if stride2:
        oy = ((i % 2 == 1) & (i < Wf - 1)).astype(F32)
    else:
        oy = ((i > 0) & (i < Wf - 1)).astype(F32)
    sel = oy[:, None] * oy[None, :]
    return jnp.pad(sel.reshape(-1), (G, G))[None]


def kernel(voxel_features, voxel_coords, batch_size, params):
    p = params
    feats = voxel_features.astype(F32)
    zv = voxel_coords[:, 1]
    fv = 83 + (voxel_coords[:, 2] + 1) * 82 + (voxel_coords[:, 3] + 1)
    M1 = 82 * 82 + 2 * 83

    npad = SC_NVOX - feats.shape[0]
    r = (SC_OFF + zv * M1 + fv).astype(jnp.int32)
    idx0 = jnp.concatenate([jnp.where(r < SC_ZONE, r, 0),
                            jnp.zeros((npad,), jnp.int32)])
    dump1 = jnp.int32(SC_OFF + ZD * M1 + 6)
    idx1 = jnp.concatenate([jnp.where(r >= SC_ZONE, r, dump1),
                            jnp.full((npad,), dump1, jnp.int32)])
    rows = jnp.concatenate(
        [feats, jnp.ones((feats.shape[0], 1), F32),
         jnp.zeros((feats.shape[0], 123), F32)], axis=1)
    rows = jnp.concatenate([rows, jnp.zeros((npad, 128), F32)], axis=0)
    table = _sc_scatter(rows, idx0.reshape(16, SC_PER // 128, 128),
                        idx1.reshape(16, SC_PER // 128, 128),
                        jnp.zeros((SC_ZPER, 128), F32))
    vol = table[SC_OFF:SC_OFF + ZD * M1, 0:8].reshape(ZD, M1, 8)
    vol = vol.transpose(0, 2, 1)
    dense = vol[:, 0:4]
    mask = vol[:, 4:5]

    wt = lambda w: jnp.transpose(w, (4, 0, 1, 2, 3)).reshape(w.shape[4], -1)
    int82, int42, int12 = _selmask(82, False), _selmask(42, False), \
        _selmask(12, False)
    odd82, odd42, odd22 = _selmask(82, True), _selmask(42, True), \
        _selmask(22, True)

    y, nm, part = _conv_layer(dense, mask, wt(p['win']), Zi=ZD, Zo=ZD, s=1,
                              zoff=-1, Wf=82, kyx=3, act=False, subm=True)
    sc, sh = _finalize(part, p['gin'], p['bin'])
    y, nm, part = _conv_layer(y, nm, wt(p['w1']), Zi=ZD, Zo=ZD, s=1,
                              zoff=-1, Wf=82, kyx=3, act=True, subm=False,
                              sc=sc, sh=sh, sel=int82)
    sc, sh = _finalize(part, p['g1'], p['b1'])
    y, nm, part = _conv_layer(y, nm, wt(p['w2a']), Zi=ZD, Zo=21, s=2,
                              zoff=-1, Wf=82, kyx=3, act=True, subm=False,
                              sc=sc, sh=sh, sel=odd82)
    y, nm = _compact(y, nm, 82, 42)
    sc, sh = _finalize(part, p['g2a'], p['b2a'])
    y, nm, part = _conv_layer(y, nm, wt(p['w2b']), Zi=21, Zo=21, s=1,
                              zoff=-1, Wf=42, kyx=3, act=True, subm=True,
                              sc=sc, sh=sh)
    sc, sh = _finalize(part, p['g2b'], p['b2b'])
    y, nm, part = _conv_layer(y, nm, wt(p['w2c']), Zi=21, Zo=21, s=1,
                              zoff=-1, Wf=42, kyx=3, act=True, subm=False,
                              sc=sc, sh=sh, sel=int42)
    sc, sh = _finalize(part, p['g2c'], p['b2c'])
    y, nm, part = _conv_layer(y, nm, wt(p['w3a']), Zi=21, Zo=11, s=2,
                              zoff=-1, Wf=42, kyx=3, act=True, subm=False,
                              sc=sc, sh=sh, sel=odd42)
    y, nm = _compact(y, nm, 42, 22)
    sc, sh = _finalize(part, p['g3a'], p['b3a'])
    y, nm, part = _conv_layer(y, nm, wt(p['w3b']), Zi=11, Zo=11, s=1,
                              zoff=-1, Wf=22, kyx=3, act=True, subm=True,
                              sc=sc, sh=sh)
    sc, sh = _finalize(part, p['g3b'], p['b3b'])
    y, nm, part = _conv_layer(y, nm, wt(p['w3c']), Zi=11, Zo=11, s=1,
                              zoff=-1, Wf=22, kyx=3, act=True, subm=True,
                              sc=sc, sh=sh)
    sc, sh = _finalize(part, p['g3c'], p['b3c'])
    y, nm, part = _conv_layer(y, nm, wt(p['w4a']), Zi=11, Zo=5, s=2,
                              zoff=0, Wf=22, kyx=3, act=True, subm=False,
                              sc=sc, sh=sh, sel=odd22)
    y, nm = _compact(y, nm, 22, 12)
    sc, sh = _finalize(part, p['g4a'], p['b4a'])
    y, nm, part = _conv_layer(y, nm, wt(p['w4b']), Zi=5, Zo=5, s=1,
                              zoff=-1, Wf=12, kyx=3, act=True, subm=True,
                              sc=sc, sh=sh)
    sc, sh = _finalize(part, p['g4b'], p['b4b'])
    y, nm, part = _conv_layer(y, nm, wt(p['w4c']), Zi=5, Zo=5, s=1,
                              zoff=-1, Wf=12, kyx=3, act=True, subm=True,
                              sc=sc, sh=sh)
    sc, sh = _finalize(part, p['g4c'], p['b4c'])
    y, nm, part = _conv_layer(y, nm, wt(p['wout']), Zi=5, Zo=2, s=2,
                              zoff=0, Wf=12, kyx=1, act=True, subm=False,
                              sc=sc, sh=sh, sel=int12)
    sc, sh = _finalize(part, p['gout'], p['bout'])
    a = _final_apply(y, nm, sc, sh)

    core = a[:, :, 13:13 + 144].reshape(2, 128, 12, 12)[:, :, 1:11, 1:11]
    return core.transpose(0, 2, 3, 1)[None]


# speedup vs baseline: 5.3303x; 5.3303x over previous
import functools

import jax
import jax.numpy as jnp
from jax import lax
from jax.experimental import pallas as pl
from jax.experimental.pallas import tpu as pltpu
from jax.experimental.pallas import tpu_sc as plsc

ZD, YD, XD = 41, 80, 80
EPS = 1e-3
F32 = jnp.float32

SC_ROWS = 282624
SC_NVOX = 40960
SC_PER = 1280
SC_DUMP = 282496


def _sc_scatter(rows, idx, tab_ref):
    mesh = plsc.VectorSubcoreMesh(core_axis_name="c", subcore_axis_name="s",
                                  num_cores=2)

    @functools.partial(
        pl.kernel, mesh=mesh,
        out_type=(),
        scratch_types=[
            pltpu.VMEM((SC_PER // 128, 128), jnp.int32),
            pltpu.VMEM((256, 128), F32),
            pltpu.SemaphoreType.DMA,
        ],
    )
    def k(rows_hbm, i_hbm, tab_hbm, idx_v, rows_v, sem):
        c = lax.axis_index("c")
        s = lax.axis_index("s")
        w = s * 2 + c
        pltpu.sync_copy(i_hbm.at[w], idx_v)
        for t in range(SC_PER // 256):
            pltpu.sync_copy(
                rows_hbm.at[pl.ds(w * SC_PER + t * 256, 256)], rows_v)
            c0 = pltpu.async_copy(rows_v.at[pl.ds(0, 128)],
                                  tab_hbm.at[idx_v.at[2 * t]], sem)
            c1 = pltpu.async_copy(rows_v.at[pl.ds(128, 128)],
                                  tab_hbm.at[idx_v.at[2 * t + 1]], sem)
            c0.wait()
            c1.wait()

    k(rows, idx, tab_ref)


def _conv_layer(x, m, wT, *, Zi, Zo, s, zoff, Wf, kyx, act, subm,
                sc=None, sh=None, sel=None):
    G = Wf + 1
    P = Wf * Wf
    M = P + 2 * G
    _, Ci, _ = x.shape
    Co = wT.shape[0]
    taps = [(dy, dx) for dy in range(kyx) for dx in range(kyx)]

    def body(*refs):
        xr = refs[0:3]
        mr = refs[3:6]
        i = 6
        if act:
            scr, shr = refs[i], refs[i + 1]
            i += 2
        if sel is not None:
            selr = refs[i]
            i += 1
        wr = refs[i]
        yref, nmref, pref = refs[i + 1:i + 4]

        zi = pl.program_id(0)
        mc = jnp.zeros((1, P), F32)
        m_center = None
        sls = []
        for dz in range(3):
            vz = s * zi + zoff + dz
            valid = jnp.where((vz >= 0) & (vz < Zi), 1.0, 0.0)
            mv = mr[dz][0] * valid
            if dz == 1:
                m_center = mr[dz][0]
            xv = xr[dz][0]
            if act:
                av = jnp.maximum(xv * scr[...] + shr[...], 0.0) * mv
            else:
                av = xv * valid
            for dy, dx in taps:
                o = (dy - 1) * Wf + (dx - 1) if kyx == 3 else 0
                sls.append(jax.lax.slice(av, (0, G + o), (Ci, G + o + P)))
                if not subm:
                    mc = mc + jax.lax.slice(mv, (0, G + o), (1, G + o + P))
        rows = sls[0] if len(sls) == 1 else jnp.concatenate(sls, axis=0)
        acc = jnp.dot(wr[...], rows, preferred_element_type=F32)
        if subm:
            nm_core = jax.lax.slice(m_center, (0, G), (1, G + P))
            nmfull = m_center
        else:
            nm_core = (mc > 0.5).astype(F32)
            nm_core = nm_core * jax.lax.slice(selr[...], (0, G), (1, G + P))
            zg = jnp.zeros((1, G), F32)
            nmfull = jnp.concatenate([zg, nm_core, zg], axis=1)
        zgc = jnp.zeros((Co, G), F32)
        yref[0] = jnp.concatenate([zgc, acc, zgc], axis=1)
        nmref[0] = nmfull

        nm_eff = nm_core
        s1 = jnp.sum(acc * nm_eff, axis=1, keepdims=True)
        s2 = jnp.sum(acc * acc * nm_eff, axis=1, keepdims=True)
        cm = jnp.zeros((Co, 1), F32) + jnp.sum(nm_eff)
        blk = jnp.concatenate([s1, s2, cm, jnp.zeros((Co, 5), F32)], axis=1)
        if Co < 128:
            blk = jnp.concatenate([blk, jnp.zeros((128 - Co, 8), F32)],
                                  axis=0)

        @pl.when(zi == 0)
        def _():
            pref[...] = blk

        @pl.when(zi != 0)
        def _():
            pref[...] = pref[...] + blk

    def xmap(d):
        return lambda z: (jnp.clip(s * z + zoff + d, 0, Zi - 1), 0, 0)

    def mmap(d):
        return lambda z: (jnp.clip(s * z + zoff + d, 0, Zi - 1), 0, 0)

    in_specs = [pl.BlockSpec((1, Ci, M), xmap(d)) for d in range(3)]
    in_specs += [pl.BlockSpec((1, 1, M), mmap(d)) for d in range(3)]
    args = [x, x, x, m, m, m]
    if act:
        in_specs += [pl.BlockSpec((Ci, 1), lambda z: (0, 0))] * 2
        args += [sc, sh]
    if sel is not None:
        in_specs += [pl.BlockSpec((1, M), lambda z: (0, 0))]
        args += [sel]
    in_specs += [pl.BlockSpec(wT.shape, lambda z: (0, 0))]
    args += [wT]

    return pl.pallas_call(
        body,
        grid=(Zo,),
        in_specs=in_specs,
        out_specs=[
            pl.BlockSpec((1, Co, M), lambda z: (z, 0, 0)),
            pl.BlockSpec((1, 1, M), lambda z: (z, 0, 0)),
            pl.BlockSpec((128, 8), lambda z: (0, 0)),
        ],
        out_shape=[
            jax.ShapeDtypeStruct((Zo, Co, M), F32),
            jax.ShapeDtypeStruct((Zo, 1, M), F32),
            jax.ShapeDtypeStruct((128, 8), F32),
        ],
    )(*args)


def _final_apply(y, nm, sc, sh):
    def body(yr, mr, scr, shr, aref):
        a = jnp.maximum(yr[...] * scr[...] + shr[...], 0.0)
        aref[...] = a * mr[...]

    return pl.pallas_call(
        body,
        out_shape=jax.ShapeDtypeStruct(y.shape, F32),
    )(y, nm, sc, sh)


def _finalize(part, g, b):
    Co = g.shape[0]
    cnt = jnp.maximum(part[0, 2], 1.0)
    mean = part[:Co, 0] / cnt
    var = jnp.maximum(part[:Co, 1] / cnt - mean * mean, 0.0)
    scale = g * jax.lax.rsqrt(var + EPS)
    shift = b - mean * scale
    return scale[:, None], shift[:, None]


def _compact(yf, nmf, Whi, Wlo):
    Zo, Co, _ = yf.shape
    Ghi, Phi = Whi + 1, Whi * Whi
    Glo = Wlo + 1
    core = yf[:, :, Ghi:Ghi + Phi].reshape(Zo, Co, Whi, Whi)
    sub = core[:, :, 1:Whi - 1:2, 1:Whi - 1:2]
    y = jnp.pad(sub, ((0, 0), (0, 0), (1, 1), (1, 1)))
    y = y.reshape(Zo, Co, Wlo * Wlo)
    y = jnp.pad(y, ((0, 0), (0, 0), (Glo, Glo)))
    mcore = nmf[:, 0, Ghi:Ghi + Phi].reshape(Zo, Whi, Whi)
    msub = mcore[:, 1:Whi - 1:2, 1:Whi - 1:2]
    nm = jnp.pad(msub, ((0, 0), (1, 1), (1, 1))).reshape(Zo, 1, Wlo * Wlo)
    nm = jnp.pad(nm, ((0, 0), (0, 0), (Glo, Glo)))
    return y, nm


def _selmask(Wf, stride2):
    G = Wf + 1
    i = jnp.arange(Wf)
    if stride2:
        oy = ((i % 2 == 1) & (i < Wf - 1)).astype(F32)
    else:
        oy = ((i > 0) & (i < Wf - 1)).astype(F32)
    sel = oy[:, None] * oy[None, :]
    return jnp.pad(sel.reshape(-1), (G, G))[None]


def kernel(voxel_features, voxel_coords, batch_size, params):
    p = params
    feats = voxel_features.astype(F32)
    zv = voxel_coords[:, 1]
    fv = 83 + (voxel_coords[:, 2] + 1) * 82 + (voxel_coords[:, 3] + 1)
    M1 = 82 * 82 + 2 * 83

    npad = SC_NVOX - feats.shape[0]
    r = (zv * M1 + fv).astype(jnp.int32)
    idx = jnp.concatenate([r, jnp.full((npad,), SC_DUMP, jnp.int32)])
    rows = jnp.concatenate(
        [feats, jnp.ones((feats.shape[0], 1), F32),
         jnp.zeros((feats.shape[0], 123), F32)], axis=1)
    rows = jnp.concatenate([rows, jnp.zeros((npad, 128), F32)], axis=0)
    tab = jax.new_ref(jnp.zeros((SC_ROWS, 128), F32))
    _sc_scatter(rows, idx.reshape(32, SC_PER // 128, 128), tab)
    vol = tab[...][0:ZD * M1, 0:8].reshape(ZD, M1, 8)
    vol = vol.transpose(0, 2, 1)
    dense = vol[:, 0:4]
    mask = vol[:, 4:5]

    wt = lambda w: jnp.transpose(w, (4, 0, 1, 2, 3)).reshape(w.shape[4], -1)
    int82, int42, int12 = _selmask(82, False), _selmask(42, False), \
        _selmask(12, False)
    odd82, odd42, odd22 = _selmask(82, True), _selmask(42, True), \
        _selmask(22, True)

    y, nm, part = _conv_layer(dense, mask, wt(p['win']), Zi=ZD, Zo=ZD, s=1,
                              zoff=-1, Wf=82, kyx=3, act=False, subm=True)
    sc, sh = _finalize(part, p['gin'], p['bin'])
    y, nm, part = _conv_layer(y, nm, wt(p['w1']), Zi=ZD, Zo=ZD, s=1,
                              zoff=-1, Wf=82, kyx=3, act=True, subm=False,
                              sc=sc, sh=sh, sel=int82)
    sc, sh = _finalize(part, p['g1'], p['b1'])
    y, nm, part = _conv_layer(y, nm, wt(p['w2a']), Zi=ZD, Zo=21, s=2,
                              zoff=-1, Wf=82, kyx=3, act=True, subm=False,
                              sc=sc, sh=sh, sel=odd82)
    y, nm = _compact(y, nm, 82, 42)
    sc, sh = _finalize(part, p['g2a'], p['b2a'])
    y, nm, part = _conv_layer(y, nm, wt(p['w2b']), Zi=21, Zo=21, s=1,
                              zoff=-1, Wf=42, kyx=3, act=True, subm=True,
                              sc=sc, sh=sh)
    sc, sh = _finalize(part, p['g2b'], p['b2b'])
    y, nm, part = _conv_layer(y, nm, wt(p['w2c']), Zi=21, Zo=21, s=1,
                              zoff=-1, Wf=42, kyx=3, act=True, subm=False,
                              sc=sc, sh=sh, sel=int42)
    sc, sh = _finalize(part, p['g2c'], p['b2c'])
    y, nm, part = _conv_layer(y, nm, wt(p['w3a']), Zi=21, Zo=11, s=2,
                              zoff=-1, Wf=42, kyx=3, act=True, subm=False,
                              sc=sc, sh=sh, sel=odd42)
    y, nm = _compact(y, nm, 42, 22)
    sc, sh = _finalize(part, p['g3a'], p['b3a'])
    y, nm, part = _conv_layer(y, nm, wt(p['w3b']), Zi=11, Zo=11, s=1,
                              zoff=-1, Wf=22, kyx=3, act=True, subm=True,
                              sc=sc, sh=sh)
    sc, sh = _finalize(part, p['g3b'], p['b3b'])
    y, nm, part = _conv_layer(y, nm, wt(p['w3c']), Zi=11, Zo=11, s=1,
                              zoff=-1, Wf=22, kyx=3, act=True, subm=True,
                              sc=sc, sh=sh)
    sc, sh = _finalize(part, p['g3c'], p['b3c'])
    y, nm, part = _conv_layer(y, nm, wt(p['w4a']), Zi=11, Zo=5, s=2,
                              zoff=0, Wf=22, kyx=3, act=True, subm=False,
                              sc=sc, sh=sh, sel=odd22)
    y, nm = _compact(y, nm, 22, 12)
    sc, sh = _finalize(part, p['g4a'], p['b4a'])
    y, nm, part = _conv_layer(y, nm, wt(p['w4b']), Zi=5, Zo=5, s=1,
                              zoff=-1, Wf=12, kyx=3, act=True, subm=True,
                              sc=sc, sh=sh)
    sc, sh = _finalize(part, p['g4b'], p['b4b'])
    y, nm, part = _conv_layer(y, nm, wt(p['w4c']), Zi=5, Zo=5, s=1,
                              zoff=-1, Wf=12, kyx=3, act=True, subm=True,
                              sc=sc, sh=sh)
    sc, sh = _finalize(part, p['g4c'], p['b4c'])
    y, nm, part = _conv_layer(y, nm, wt(p['wout']), Zi=5, Zo=2, s=2,
                              zoff=0, Wf=12, kyx=1, act=True, subm=False,
                              sc=sc, sh=sh, sel=int12)
    sc, sh = _finalize(part, p['gout'], p['bout'])
    a = _final_apply(y, nm, sc, sh)

    core = a[:, :, 13:13 + 144].reshape(2, 128, 12, 12)[:, :, 1:11, 1:11]
    return core.transpose(0, 2, 3, 1)[None]
